# Initial kernel scaffold; baseline (speedup 1.0000x reference)
import jax
import jax.numpy as jnp
from jax.experimental import pallas as pl


def kernel(input, weights):
    raise NotImplementedError("write your pallas kernel here")


# speedup vs baseline: 2.0583x; 2.0583x over previous
import functools

import jax
import jax.numpy as jnp
from jax import lax
from jax.experimental import pallas as pl
from jax.experimental.pallas import tpu as pltpu
from jax.experimental.pallas import tpu_sc as plsc

PADDING_IDX = 1
LANES = 16

_NC = 2
_NS = 16
_NW = _NC * _NS


def _make_sc_lookup(bsz, seq, vocab_rows, d):
    tok_total = bsz * seq
    bpw = tok_total // _NW
    assert tok_total % _NW == 0 and seq % bpw == 0
    ch = 64
    nchunks = bpw // ch
    mesh = plsc.VectorSubcoreMesh(core_axis_name="c", subcore_axis_name="s")

    @functools.partial(
        pl.kernel,
        mesh=mesh,
        out_type=jax.ShapeDtypeStruct((tok_total, d), jnp.float32),
        scratch_types=[
            pltpu.VMEM((bpw,), jnp.int32),
            pltpu.VMEM((ch,), jnp.int32),
            pltpu.VMEM((ch, d), jnp.float32),
            pltpu.SemaphoreType.DMA,
        ],
    )
    def lookup(tok_hbm, w_hbm, out_hbm, tok_v, idx_v, rows_v, sem):
        wid = lax.axis_index("s") * _NC + lax.axis_index("c")
        base = wid * bpw
        col0 = (base % seq)
        pltpu.sync_copy(tok_hbm.at[pl.ds(base, bpw)], tok_v)
        lane = lax.broadcasted_iota(jnp.int32, (LANES,), 0)
        for c in range(nchunks):
            for g in range(ch // LANES):
                off = c * ch + g * LANES
                t = tok_v[pl.ds(off, LANES)]
                col = lane + (col0 + off)
                p = jnp.where(t != PADDING_IDX, col + (PADDING_IDX + 1),
                              PADDING_IDX)
                idx_v[pl.ds(g * LANES, LANES)] = p
            pltpu.async_copy(w_hbm.at[idx_v], rows_v, sem).wait()
            pltpu.sync_copy(rows_v, out_hbm.at[pl.ds(base + c * ch, ch)])

    return lookup


def kernel(input, weights):
    bsz, seq = input.shape
    vocab_rows, d = weights.shape
    lookup = _make_sc_lookup(bsz, seq, vocab_rows, d)
    out = lookup(input.reshape(-1), weights)
    return out.reshape(bsz, seq, d)
